# Initial kernel scaffold; baseline (speedup 1.0000x reference)
import jax
import jax.numpy as jnp
from jax.experimental import pallas as pl


def kernel(x, enc_W1, enc_b1, enc_W2, enc_b2, codebook, dec_W1, dec_b1, dec_W2, dec_b2):
    raise NotImplementedError("write your pallas kernel here")


# speedup vs baseline: 1.3802x; 1.3802x over previous
import jax
import jax.numpy as jnp
from jax.experimental import pallas as pl
from jax.experimental.pallas import tpu as pltpu

_B, _S, _F = 4, 256, 256
_L = 64
_K = 1024
_T = _B * _S

_HI = jax.lax.Precision.HIGHEST


_TT = 128


def _encvq_body(x_ref, w1_ref, b1_ref, w2_ref, b2_ref, cb_ref, cbt_ref,
                eps_ref, mean_ref, logvar_ref, zq_ref, vq_ref):
    i = pl.program_id(0)
    x = x_ref[...]
    h = jnp.maximum(
        jnp.dot(x, w1_ref[...], preferred_element_type=jnp.float32,
                precision=_HI) + b1_ref[...], 0.0)
    enc = jnp.dot(h, w2_ref[...], preferred_element_type=jnp.float32,
                  precision=_HI) + b2_ref[...]
    mean = enc[:, :_L]
    logvar = enc[:, _L:]
    z_e = mean + jnp.exp(0.5 * logvar) * eps_ref[...]

    cb = cb_ref[...]
    cbt = cbt_ref[...]
    csq = jnp.sum(cbt * cbt, axis=0, keepdims=True)
    cross = jnp.dot(z_e, cbt, preferred_element_type=jnp.float32,
                    precision=_HI)
    dist = csq - 2.0 * cross
    m = jnp.min(dist, axis=1, keepdims=True)
    lane = jax.lax.broadcasted_iota(jnp.int32, (_TT, _K), 1)
    ids = jnp.min(jnp.where(dist <= m, lane, _K), axis=1, keepdims=True)
    onehot = (lane == ids).astype(jnp.float32)
    z_q = jnp.dot(onehot, cb, preferred_element_type=jnp.float32,
                  precision=_HI)

    mean_ref[...] = mean
    logvar_ref[...] = logvar
    zq_ref[...] = z_q
    d = z_e - z_q
    part = jnp.sum(jnp.sum(d * d, axis=1, keepdims=True),
                   axis=0, keepdims=True) / (_T * _L)

    @pl.when(i == 0)
    def _init():
        vq_ref[...] = part

    @pl.when(i > 0)
    def _acc():
        vq_ref[...] = vq_ref[...] + part


def _dec1_body(flat_ref, w1_ref, b1_ref, out_ref):
    k = pl.program_id(0)
    nk = pl.num_programs(0)
    part = jnp.dot(flat_ref[...], w1_ref[...],
                   preferred_element_type=jnp.float32, precision=_HI)

    @pl.when(k == 0)
    def _init():
        out_ref[...] = part

    @pl.when(k > 0)
    def _acc():
        out_ref[...] = out_ref[...] + part

    @pl.when(k == nk - 1)
    def _fin():
        out_ref[...] = jnp.maximum(out_ref[...] + b1_ref[...], 0.0)


def _dec2_body(d1_ref, w2_ref, b2_ref, out_ref):
    t = jnp.dot(d1_ref[...], w2_ref[...],
                preferred_element_type=jnp.float32, precision=_HI) + b2_ref[...]
    out_ref[...] = jnp.maximum(t, 0.0) + jnp.log1p(jnp.exp(-jnp.abs(t)))


def kernel(x, enc_W1, enc_b1, enc_W2, enc_b2, codebook,
           dec_W1, dec_b1, dec_W2, dec_b2):
    x2 = x.reshape(_T, _F)
    eps = jax.random.normal(jax.random.key(42), (_B, _S, _L),
                            dtype=jnp.float32).reshape(_T, _L)

    nts = _T // _TT
    mean, logvar, z_q, vq = pl.pallas_call(
        _encvq_body,
        grid=(nts,),
        in_specs=[
            pl.BlockSpec((_TT, _F), lambda i: (i, 0)),
            pl.BlockSpec((_F, 512), lambda i: (0, 0)),
            pl.BlockSpec((1, 512), lambda i: (0, 0)),
            pl.BlockSpec((512, 2 * _L), lambda i: (0, 0)),
            pl.BlockSpec((1, 2 * _L), lambda i: (0, 0)),
            pl.BlockSpec((_K, _L), lambda i: (0, 0)),
            pl.BlockSpec((_L, _K), lambda i: (0, 0)),
            pl.BlockSpec((_TT, _L), lambda i: (i, 0)),
        ],
        out_specs=(
            pl.BlockSpec((_TT, _L), lambda i: (i, 0)),
            pl.BlockSpec((_TT, _L), lambda i: (i, 0)),
            pl.BlockSpec((_TT, _L), lambda i: (i, 0)),
            pl.BlockSpec((1, 1), lambda i: (0, 0)),
        ),
        out_shape=(
            jax.ShapeDtypeStruct((_T, _L), jnp.float32),
            jax.ShapeDtypeStruct((_T, _L), jnp.float32),
            jax.ShapeDtypeStruct((_T, _L), jnp.float32),
            jax.ShapeDtypeStruct((1, 1), jnp.float32),
        ),
    )(x2, enc_W1, enc_b1.reshape(1, 512), enc_W2, enc_b2.reshape(1, 2 * _L),
      codebook, codebook.T, eps)

    flat = z_q.reshape(_B, _S * _L)

    kt = 8
    kc = (_S * _L) // kt
    d1 = pl.pallas_call(
        _dec1_body,
        grid=(kt,),
        in_specs=[
            pl.BlockSpec((_B, kc), lambda k: (0, k)),
            pl.BlockSpec((kc, 512), lambda k: (k, 0)),
            pl.BlockSpec((1, 512), lambda k: (0, 0)),
        ],
        out_specs=pl.BlockSpec((_B, 512), lambda k: (0, 0)),
        out_shape=jax.ShapeDtypeStruct((_B, 512), jnp.float32),
    )(flat, dec_W1, dec_b1.reshape(1, 512))

    nt = 16
    nc = (_S * _F) // nt
    rec = pl.pallas_call(
        _dec2_body,
        grid=(nt,),
        in_specs=[
            pl.BlockSpec((_B, 512), lambda n: (0, 0)),
            pl.BlockSpec((512, nc), lambda n: (0, n)),
            pl.BlockSpec((1, nc), lambda n: (0, n)),
        ],
        out_specs=pl.BlockSpec((_B, nc), lambda n: (0, n)),
        out_shape=jax.ShapeDtypeStruct((_B, _S * _F), jnp.float32),
    )(d1, dec_W2, dec_b2.reshape(1, _S * _F))

    reconstructed = rec.reshape(_B, _S, _F)
    return (reconstructed,
            mean.reshape(_B, _S, _L),
            logvar.reshape(_B, _S, _L),
            vq[0, 0])


# speedup vs baseline: 1.8057x; 1.3083x over previous
import jax
import jax.numpy as jnp
from jax.experimental import pallas as pl
from jax.experimental.pallas import tpu as pltpu

_B, _S, _F = 4, 256, 256
_L = 64
_K = 1024
_T = _B * _S

_HI = jax.lax.Precision.HIGHEST


_TT = 128


def _encvq_body(x_ref, w1_ref, b1_ref, w2_ref, b2_ref, cb_ref, cbt_ref,
                eps_ref, mean_ref, logvar_ref, zq_ref, vq_ref):
    i = pl.program_id(0)
    x = x_ref[...]
    h = jnp.maximum(
        jnp.dot(x, w1_ref[...], preferred_element_type=jnp.float32,
                precision=_HI) + b1_ref[...], 0.0)
    enc = jnp.dot(h, w2_ref[...], preferred_element_type=jnp.float32,
                  precision=_HI) + b2_ref[...]
    mean = enc[:, :_L]
    logvar = enc[:, _L:]
    z_e = mean + jnp.exp(0.5 * logvar) * eps_ref[...]

    cb = cb_ref[...]
    cbt = cbt_ref[...]
    csq = jnp.sum(cbt * cbt, axis=0, keepdims=True)
    cross = jnp.dot(z_e, cbt, preferred_element_type=jnp.float32,
                    precision=_HI)
    dist = csq - 2.0 * cross
    m = jnp.min(dist, axis=1, keepdims=True)
    lane = jax.lax.broadcasted_iota(jnp.int32, (_TT, _K), 1)
    ids = jnp.min(jnp.where(dist <= m, lane, _K), axis=1, keepdims=True)
    onehot = (lane == ids).astype(jnp.float32)
    z_q = jnp.dot(onehot, cb, preferred_element_type=jnp.float32,
                  precision=_HI)

    mean_ref[...] = mean
    logvar_ref[...] = logvar
    zq_ref[...] = z_q
    d = z_e - z_q
    part = jnp.sum(jnp.sum(d * d, axis=1, keepdims=True),
                   axis=0, keepdims=True) / (_T * _L)

    @pl.when(i == 0)
    def _init():
        vq_ref[...] = part

    @pl.when(i > 0)
    def _acc():
        vq_ref[...] = vq_ref[...] + part


def _dec1_body(flat_ref, w1_ref, b1_ref, out_ref):
    k = pl.program_id(0)
    nk = pl.num_programs(0)
    part = jnp.dot(flat_ref[...].astype(jnp.bfloat16),
                   w1_ref[...].astype(jnp.bfloat16),
                   preferred_element_type=jnp.float32)

    @pl.when(k == 0)
    def _init():
        out_ref[...] = part

    @pl.when(k > 0)
    def _acc():
        out_ref[...] = out_ref[...] + part

    @pl.when(k == nk - 1)
    def _fin():
        out_ref[...] = jnp.maximum(out_ref[...] + b1_ref[...], 0.0)


def _dec2_body(d1_ref, w2_ref, b2_ref, out_ref):
    t = jnp.dot(d1_ref[...].astype(jnp.bfloat16),
                w2_ref[...].astype(jnp.bfloat16),
                preferred_element_type=jnp.float32) + b2_ref[...]
    out_ref[...] = jnp.maximum(t, 0.0) + jnp.log1p(jnp.exp(-jnp.abs(t)))


def kernel(x, enc_W1, enc_b1, enc_W2, enc_b2, codebook,
           dec_W1, dec_b1, dec_W2, dec_b2):
    x2 = x.reshape(_T, _F)
    eps = jax.random.normal(jax.random.key(42), (_B, _S, _L),
                            dtype=jnp.float32).reshape(_T, _L)

    nts = _T // _TT
    mean, logvar, z_q, vq = pl.pallas_call(
        _encvq_body,
        grid=(nts,),
        in_specs=[
            pl.BlockSpec((_TT, _F), lambda i: (i, 0)),
            pl.BlockSpec((_F, 512), lambda i: (0, 0)),
            pl.BlockSpec((1, 512), lambda i: (0, 0)),
            pl.BlockSpec((512, 2 * _L), lambda i: (0, 0)),
            pl.BlockSpec((1, 2 * _L), lambda i: (0, 0)),
            pl.BlockSpec((_K, _L), lambda i: (0, 0)),
            pl.BlockSpec((_L, _K), lambda i: (0, 0)),
            pl.BlockSpec((_TT, _L), lambda i: (i, 0)),
        ],
        out_specs=(
            pl.BlockSpec((_TT, _L), lambda i: (i, 0)),
            pl.BlockSpec((_TT, _L), lambda i: (i, 0)),
            pl.BlockSpec((_TT, _L), lambda i: (i, 0)),
            pl.BlockSpec((1, 1), lambda i: (0, 0)),
        ),
        out_shape=(
            jax.ShapeDtypeStruct((_T, _L), jnp.float32),
            jax.ShapeDtypeStruct((_T, _L), jnp.float32),
            jax.ShapeDtypeStruct((_T, _L), jnp.float32),
            jax.ShapeDtypeStruct((1, 1), jnp.float32),
        ),
    )(x2, enc_W1, enc_b1.reshape(1, 512), enc_W2, enc_b2.reshape(1, 2 * _L),
      codebook, codebook.T, eps)

    flat = z_q.reshape(_B, _S * _L)

    kt = 8
    kc = (_S * _L) // kt
    d1 = pl.pallas_call(
        _dec1_body,
        grid=(kt,),
        in_specs=[
            pl.BlockSpec((_B, kc), lambda k: (0, k)),
            pl.BlockSpec((kc, 512), lambda k: (k, 0)),
            pl.BlockSpec((1, 512), lambda k: (0, 0)),
        ],
        out_specs=pl.BlockSpec((_B, 512), lambda k: (0, 0)),
        out_shape=jax.ShapeDtypeStruct((_B, 512), jnp.float32),
    )(flat, dec_W1, dec_b1.reshape(1, 512))

    nt = 16
    nc = (_S * _F) // nt
    rec = pl.pallas_call(
        _dec2_body,
        grid=(nt,),
        in_specs=[
            pl.BlockSpec((_B, 512), lambda n: (0, 0)),
            pl.BlockSpec((512, nc), lambda n: (0, n)),
            pl.BlockSpec((1, nc), lambda n: (0, n)),
        ],
        out_specs=pl.BlockSpec((_B, nc), lambda n: (0, n)),
        out_shape=jax.ShapeDtypeStruct((_B, _S * _F), jnp.float32),
    )(d1, dec_W2, dec_b2.reshape(1, _S * _F))

    reconstructed = rec.reshape(_B, _S, _F)
    return (reconstructed,
            mean.reshape(_B, _S, _L),
            logvar.reshape(_B, _S, _L),
            vq[0, 0])


# speedup vs baseline: 1.8238x; 1.0100x over previous
import jax
import jax.numpy as jnp
from jax.experimental import pallas as pl
from jax.experimental.pallas import tpu as pltpu

_B, _S, _F = 4, 256, 256
_L = 64
_K = 1024
_T = _B * _S

_HI = jax.lax.Precision.HIGHEST


_TT = 256


def _encvq_body(x_ref, w1_ref, b1_ref, w2_ref, b2_ref, cb_ref, cbt_ref,
                eps_ref, mean_ref, logvar_ref, zq_ref, vq_ref):
    i = pl.program_id(0)
    x = x_ref[...]
    h = jnp.maximum(
        jnp.dot(x, w1_ref[...], preferred_element_type=jnp.float32,
                precision=_HI) + b1_ref[...], 0.0)
    enc = jnp.dot(h, w2_ref[...], preferred_element_type=jnp.float32,
                  precision=_HI) + b2_ref[...]
    mean = enc[:, :_L]
    logvar = enc[:, _L:]
    z_e = mean + jnp.exp(0.5 * logvar) * eps_ref[...]

    cb = cb_ref[...]
    cbt = cbt_ref[...]
    csq = jnp.sum(cbt * cbt, axis=0, keepdims=True)
    cross = jnp.dot(z_e, cbt, preferred_element_type=jnp.float32,
                    precision=_HI)
    dist = csq - 2.0 * cross
    m = jnp.min(dist, axis=1, keepdims=True)
    lane = jax.lax.broadcasted_iota(jnp.int32, (_TT, _K), 1)
    ids = jnp.min(jnp.where(dist <= m, lane, _K), axis=1, keepdims=True)
    onehot = (lane == ids).astype(jnp.float32)
    z_q = jnp.dot(onehot, cb, preferred_element_type=jnp.float32,
                  precision=_HI)

    mean_ref[...] = mean
    logvar_ref[...] = logvar
    zq_ref[...] = z_q
    d = z_e - z_q
    part = jnp.sum(jnp.sum(d * d, axis=1, keepdims=True),
                   axis=0, keepdims=True) / (_T * _L)

    @pl.when(i == 0)
    def _init():
        vq_ref[...] = part

    @pl.when(i > 0)
    def _acc():
        vq_ref[...] = vq_ref[...] + part


_K1 = 4
_N2 = 8


def _dec_body(flat_ref, w1_ref, b1_ref, w2_ref, b2_ref, out_ref, acc_ref):
    k = pl.program_id(0)

    @pl.when(k < _K1)
    def _dec1():
        part = jnp.dot(flat_ref[...].astype(jnp.bfloat16),
                       w1_ref[...].astype(jnp.bfloat16),
                       preferred_element_type=jnp.float32)

        @pl.when(k == 0)
        def _init():
            acc_ref[...] = part

        @pl.when(k > 0)
        def _acc():
            acc_ref[...] = acc_ref[...] + part

        @pl.when(k == _K1 - 1)
        def _fin():
            acc_ref[...] = jnp.maximum(acc_ref[...] + b1_ref[...], 0.0)

    @pl.when(k >= _K1)
    def _dec2():
        t = jnp.dot(acc_ref[...].astype(jnp.bfloat16),
                    w2_ref[...].astype(jnp.bfloat16),
                    preferred_element_type=jnp.float32) + b2_ref[...]
        out_ref[...] = jnp.maximum(t, 0.0) + jnp.log1p(jnp.exp(-jnp.abs(t)))


def kernel(x, enc_W1, enc_b1, enc_W2, enc_b2, codebook,
           dec_W1, dec_b1, dec_W2, dec_b2):
    x2 = x.reshape(_T, _F)
    eps = jax.random.normal(jax.random.key(42), (_B, _S, _L),
                            dtype=jnp.float32).reshape(_T, _L)

    nts = _T // _TT
    mean, logvar, z_q, vq = pl.pallas_call(
        _encvq_body,
        grid=(nts,),
        in_specs=[
            pl.BlockSpec((_TT, _F), lambda i: (i, 0)),
            pl.BlockSpec((_F, 512), lambda i: (0, 0)),
            pl.BlockSpec((1, 512), lambda i: (0, 0)),
            pl.BlockSpec((512, 2 * _L), lambda i: (0, 0)),
            pl.BlockSpec((1, 2 * _L), lambda i: (0, 0)),
            pl.BlockSpec((_K, _L), lambda i: (0, 0)),
            pl.BlockSpec((_L, _K), lambda i: (0, 0)),
            pl.BlockSpec((_TT, _L), lambda i: (i, 0)),
        ],
        out_specs=(
            pl.BlockSpec((_TT, _L), lambda i: (i, 0)),
            pl.BlockSpec((_TT, _L), lambda i: (i, 0)),
            pl.BlockSpec((_TT, _L), lambda i: (i, 0)),
            pl.BlockSpec((1, 1), lambda i: (0, 0)),
        ),
        out_shape=(
            jax.ShapeDtypeStruct((_T, _L), jnp.float32),
            jax.ShapeDtypeStruct((_T, _L), jnp.float32),
            jax.ShapeDtypeStruct((_T, _L), jnp.float32),
            jax.ShapeDtypeStruct((1, 1), jnp.float32),
        ),
    )(x2, enc_W1, enc_b1.reshape(1, 512), enc_W2, enc_b2.reshape(1, 2 * _L),
      codebook, codebook.T, eps)

    flat = z_q.reshape(_B, _S * _L)

    kc = (_S * _L) // _K1
    nc = (_S * _F) // _N2
    rec = pl.pallas_call(
        _dec_body,
        grid=(_K1 + _N2,),
        in_specs=[
            pl.BlockSpec((_B, kc), lambda k: (0, jnp.minimum(k, _K1 - 1))),
            pl.BlockSpec((kc, 512), lambda k: (jnp.minimum(k, _K1 - 1), 0)),
            pl.BlockSpec((1, 512), lambda k: (0, 0)),
            pl.BlockSpec((512, nc), lambda k: (0, jnp.maximum(k - _K1, 0))),
            pl.BlockSpec((1, nc), lambda k: (0, jnp.maximum(k - _K1, 0))),
        ],
        out_specs=pl.BlockSpec((_B, nc), lambda k: (0, jnp.maximum(k - _K1, 0))),
        out_shape=jax.ShapeDtypeStruct((_B, _S * _F), jnp.float32),
        scratch_shapes=[pltpu.VMEM((_B, 512), jnp.float32)],
    )(flat, dec_W1, dec_b1.reshape(1, 512), dec_W2, dec_b2.reshape(1, _S * _F))

    reconstructed = rec.reshape(_B, _S, _F)
    return (reconstructed,
            mean.reshape(_B, _S, _L),
            logvar.reshape(_B, _S, _L),
            vq[0, 0])
